# baseline (speedup 1.0000x reference)
import functools
import jax
import jax.numpy as jnp
from jax import lax
from jax.experimental import pallas as pl
from jax.experimental.pallas import tpu as pltpu
from jax.experimental.pallas import tpu_sc as plsc

B = 16384
D = 128
L = 16
NC = 2
NS = 16
HALF = B // 2
BPW = HALF // (NC * NS)
CHUNK = 64
NCHUNK = BPW // CHUNK
NG = CHUNK // L


def _sc_gather_half(h_base, uid_hbm, mid_hbm, gu_t, gm_t, mu_t, mm_t,
                    gu_o, gm_o, mu_o, mm_o,
                    idx_u, idx_m,
                    buf_gu, buf_gm, buf_mu, buf_mm, sem_g, sem_w):
    c = lax.axis_index("c")
    s = lax.axis_index("s")
    base = (s * NC + c) * BPW
    ids_base = h_base + base
    pltpu.sync_copy(uid_hbm.at[pl.ds(ids_base, BPW)], idx_u)
    pltpu.sync_copy(mid_hbm.at[pl.ds(ids_base, BPW)], idx_m)

    def issue_gathers(k, sel):
        iu = idx_u.at[pl.ds(k * CHUNK, CHUNK)]
        im = idx_m.at[pl.ds(k * CHUNK, CHUNK)]
        return [pltpu.async_copy(gu_t.at[iu], buf_gu[sel], sem_g),
                pltpu.async_copy(gm_t.at[im], buf_gm[sel], sem_g),
                pltpu.async_copy(mu_t.at[iu], buf_mu[sel], sem_g),
                pltpu.async_copy(mm_t.at[im], buf_mm[sel], sem_g)]

    pend_g = issue_gathers(0, 0)
    pend_w = []
    for k in range(NCHUNK):
        sel = k % 2
        if k + 1 < NCHUNK:
            for cp in pend_w:
                cp.wait()
            pend_w = []
            pend_g_next = issue_gathers(k + 1, 1 - sel)
        for cp in pend_g:
            cp.wait()
        if k + 1 < NCHUNK:
            pend_g = pend_g_next
        rows = pl.ds(base + k * CHUNK, CHUNK)
        pend_w.append(pltpu.async_copy(buf_gu[sel], gu_o.at[rows], sem_w))
        pend_w.append(pltpu.async_copy(buf_gm[sel], gm_o.at[rows], sem_w))
        pend_w.append(pltpu.async_copy(buf_mu[sel], mu_o.at[rows], sem_w))
        pend_w.append(pltpu.async_copy(buf_mm[sel], mm_o.at[rows], sem_w))

    for cp in pend_w:
        cp.wait()


@functools.partial(jax.jit, static_argnums=0)
def _sc_gather(h, user_ids, movie_ids, gu_t, gm_t, mu_t, mm_t):
    mesh = plsc.VectorSubcoreMesh(core_axis_name="c", subcore_axis_name="s",
                                  num_cores=NC, num_subcores=NS)
    row = jax.ShapeDtypeStruct((HALF, D), jnp.float32)
    dbuf = [pltpu.VMEM((CHUNK, D), jnp.float32)] * 2
    return pl.kernel(
        functools.partial(_sc_gather_half, h * HALF),
        out_type=[row, row, row, row],
        mesh=mesh,
        scratch_types=[
            pltpu.VMEM((BPW,), jnp.int32),
            pltpu.VMEM((BPW,), jnp.int32),
            dbuf, dbuf, dbuf, dbuf,
            pltpu.SemaphoreType.DMA,
            pltpu.SemaphoreType.DMA,
        ],
    )(user_ids, movie_ids, gu_t, gm_t, mu_t, mm_t)


BT = 8192


def _tc_dense_body(gu, gm, mu, mm, w1t, b1, w2t, b2, woutt, bb, out):
    h = jnp.concatenate([mu[...], mm[...]], axis=1)
    h1 = jnp.maximum(
        jnp.dot(h, w1t[...], preferred_element_type=jnp.float32)
        + b1[...], 0.0)
    h2 = jnp.maximum(
        jnp.dot(h1, w2t[...], preferred_element_type=jnp.float32)
        + b2[...], 0.0)
    cat = jnp.concatenate([gu[...] * gm[...], h2], axis=1)
    o = jnp.dot(cat, woutt[...], preferred_element_type=jnp.float32)
    out[...] = o[:, 0] + bb[0]


@jax.jit
def _tc_dense(gu, gm, mu, mm, w1t, b1, w2t, b2, woutt, bb):
    row_spec = pl.BlockSpec((BT, D), lambda i: (i, 0))

    def full(shape):
        return pl.BlockSpec(shape, lambda i: (0, 0))

    grid = (HALF // BT,)
    return pl.pallas_call(
        _tc_dense_body,
        grid=grid,
        in_specs=[row_spec, row_spec, row_spec, row_spec,
                  full((2 * D, 64)), full((1, 64)),
                  full((64, D)), full((1, D)), full((2 * D, 1)),
                  pl.BlockSpec(memory_space=pltpu.SMEM)],
        out_specs=pl.BlockSpec((BT,), lambda i: (i,)),
        out_shape=jax.ShapeDtypeStruct((HALF,), jnp.float32),
    )(gu, gm, mu, mm, w1t, b1, w2t, b2, woutt, bb)


def kernel(user_ids, movie_ids, gmf_user_table, gmf_movie_table,
           mlp_user_table, mlp_movie_table, W1, b1, W2, b2, Wout, bout):
    w1t = W1.T
    w2t = W2.T
    woutt = Wout.T

    outs = []
    for h in range(2):
        gu, gm, mu, mm = _sc_gather(h, user_ids, movie_ids, gmf_user_table,
                                    gmf_movie_table, mlp_user_table,
                                    mlp_movie_table)
        outs.append(_tc_dense(gu, gm, mu, mm, w1t, b1.reshape(1, -1),
                              w2t, b2.reshape(1, -1), woutt, bout))
    return jnp.concatenate(outs, axis=0)


# speedup vs baseline: 1.0214x; 1.0214x over previous
import functools
import jax
import jax.numpy as jnp
from jax import lax
from jax.experimental import pallas as pl
from jax.experimental.pallas import tpu as pltpu
from jax.experimental.pallas import tpu_sc as plsc

B = 16384
D = 128
L = 16
NC = 2
NS = 16
HALF = B // 2
BPW = HALF // (NC * NS)
CHUNK = 64
NCHUNK = BPW // CHUNK
NG = CHUNK // L


def _sc_gather_half(h_base, uid_hbm, mid_hbm, gu_t, gm_t, mu_t, mm_t,
                    gu_o, gm_o, mu_o, mm_o,
                    idx_u, idx_m,
                    buf_gu, buf_gm, buf_mu, buf_mm, sem_g, sem_w):
    c = lax.axis_index("c")
    s = lax.axis_index("s")
    base = (s * NC + c) * BPW
    ids_base = h_base + base
    pltpu.sync_copy(uid_hbm.at[pl.ds(ids_base, BPW)], idx_u)
    pltpu.sync_copy(mid_hbm.at[pl.ds(ids_base, BPW)], idx_m)

    def issue_gathers(k, sel):
        iu = idx_u.at[pl.ds(k * CHUNK, CHUNK)]
        im = idx_m.at[pl.ds(k * CHUNK, CHUNK)]
        return [pltpu.async_copy(gu_t.at[iu], buf_gu[sel], sem_g),
                pltpu.async_copy(gm_t.at[im], buf_gm[sel], sem_g),
                pltpu.async_copy(mu_t.at[iu], buf_mu[sel], sem_g),
                pltpu.async_copy(mm_t.at[im], buf_mm[sel], sem_g)]

    pend_g = issue_gathers(0, 0)
    pend_w = []
    for k in range(NCHUNK):
        sel = k % 2
        if k + 1 < NCHUNK:
            for cp in pend_w:
                cp.wait()
            pend_w = []
            pend_g_next = issue_gathers(k + 1, 1 - sel)
        for cp in pend_g:
            cp.wait()
        if k + 1 < NCHUNK:
            pend_g = pend_g_next
        rows = pl.ds(base + k * CHUNK, CHUNK)
        pend_w.append(pltpu.async_copy(buf_gu[sel], gu_o.at[rows], sem_w))
        pend_w.append(pltpu.async_copy(buf_gm[sel], gm_o.at[rows], sem_w))
        pend_w.append(pltpu.async_copy(buf_mu[sel], mu_o.at[rows], sem_w))
        pend_w.append(pltpu.async_copy(buf_mm[sel], mm_o.at[rows], sem_w))

    for cp in pend_w:
        cp.wait()


@functools.partial(jax.jit, static_argnums=0)
def _sc_gather(h, user_ids, movie_ids, gu_t, gm_t, mu_t, mm_t):
    mesh = plsc.VectorSubcoreMesh(core_axis_name="c", subcore_axis_name="s",
                                  num_cores=NC, num_subcores=NS)
    row = jax.ShapeDtypeStruct((HALF, D), jnp.float32)
    dbuf = [pltpu.VMEM((CHUNK, D), jnp.float32)] * 2
    return pl.kernel(
        functools.partial(_sc_gather_half, h * HALF),
        out_type=[row, row, row, row],
        mesh=mesh,
        scratch_types=[
            pltpu.VMEM((BPW,), jnp.int32),
            pltpu.VMEM((BPW,), jnp.int32),
            dbuf, dbuf, dbuf, dbuf,
            pltpu.SemaphoreType.DMA,
            pltpu.SemaphoreType.DMA,
        ],
    )(user_ids, movie_ids, gu_t, gm_t, mu_t, mm_t)


BT = 4096


def _tc_dense_body(gu, gm, mu, mm, w1t, b1, w2t, b2, woutt, bb, out):
    h = jnp.concatenate([mu[...], mm[...]], axis=1)
    h1 = jnp.maximum(
        jnp.dot(h, w1t[...], preferred_element_type=jnp.float32)
        + b1[...], 0.0)
    h2 = jnp.maximum(
        jnp.dot(h1, w2t[...], preferred_element_type=jnp.float32)
        + b2[...], 0.0)
    cat = jnp.concatenate([gu[...] * gm[...], h2], axis=1)
    o = jnp.dot(cat, woutt[...], preferred_element_type=jnp.float32)
    out[...] = o[:, 0] + bb[0]


@jax.jit
def _tc_dense(gu, gm, mu, mm, w1t, b1, w2t, b2, woutt, bb):
    row_spec = pl.BlockSpec((BT, D), lambda i: (i, 0))

    def full(shape):
        return pl.BlockSpec(shape, lambda i: (0, 0))

    grid = (HALF // BT,)
    return pl.pallas_call(
        _tc_dense_body,
        grid=grid,
        in_specs=[row_spec, row_spec, row_spec, row_spec,
                  full((2 * D, 64)), full((1, 64)),
                  full((64, D)), full((1, D)), full((2 * D, 1)),
                  pl.BlockSpec(memory_space=pltpu.SMEM)],
        out_specs=pl.BlockSpec((BT,), lambda i: (i,)),
        out_shape=jax.ShapeDtypeStruct((HALF,), jnp.float32),
    )(gu, gm, mu, mm, w1t, b1, w2t, b2, woutt, bb)


def kernel(user_ids, movie_ids, gmf_user_table, gmf_movie_table,
           mlp_user_table, mlp_movie_table, W1, b1, W2, b2, Wout, bout):
    w1t = W1.T
    w2t = W2.T
    woutt = Wout.T

    outs = []
    for h in range(2):
        gu, gm, mu, mm = _sc_gather(h, user_ids, movie_ids, gmf_user_table,
                                    gmf_movie_table, mlp_user_table,
                                    mlp_movie_table)
        outs.append(_tc_dense(gu, gm, mu, mm, w1t, b1.reshape(1, -1),
                              w2t, b2.reshape(1, -1), woutt, bout))
    return jnp.concatenate(outs, axis=0)


# speedup vs baseline: 1.0455x; 1.0236x over previous
import functools
import jax
import jax.numpy as jnp
from jax import lax
from jax.experimental import pallas as pl
from jax.experimental.pallas import tpu as pltpu
from jax.experimental.pallas import tpu_sc as plsc

B = 16384
D = 128
L = 16
NC = 2
NS = 16
HALF = B // 2
BPW = HALF // (NC * NS)
CHUNK = 64
NCHUNK = BPW // CHUNK
NG = CHUNK // L


def _sc_gather_half(h_base, uid_hbm, mid_hbm, gu_t, gm_t, mu_t, mm_t,
                    gu_o, gm_o, mu_o, mm_o,
                    idx_u, idx_m,
                    buf_gu, buf_gm, buf_mu, buf_mm, sem_g, sem_w):
    c = lax.axis_index("c")
    s = lax.axis_index("s")
    base = (s * NC + c) * BPW
    ids_base = h_base + base
    pltpu.sync_copy(uid_hbm.at[pl.ds(ids_base, BPW)], idx_u)
    pltpu.sync_copy(mid_hbm.at[pl.ds(ids_base, BPW)], idx_m)

    def issue_gathers(k, sel):
        iu = idx_u.at[pl.ds(k * CHUNK, CHUNK)]
        im = idx_m.at[pl.ds(k * CHUNK, CHUNK)]
        return [pltpu.async_copy(gu_t.at[iu], buf_gu[sel], sem_g),
                pltpu.async_copy(gm_t.at[im], buf_gm[sel], sem_g),
                pltpu.async_copy(mu_t.at[iu], buf_mu[sel], sem_g),
                pltpu.async_copy(mm_t.at[im], buf_mm[sel], sem_g)]

    pend_g = issue_gathers(0, 0)
    pend_w = []
    for k in range(NCHUNK):
        sel = k % 2
        if k + 1 < NCHUNK:
            for cp in pend_w:
                cp.wait()
            pend_w = []
            pend_g_next = issue_gathers(k + 1, 1 - sel)
        for cp in pend_g:
            cp.wait()
        if k + 1 < NCHUNK:
            pend_g = pend_g_next
        rows = pl.ds(base + k * CHUNK, CHUNK)
        pend_w.append(pltpu.async_copy(buf_gu[sel], gu_o.at[rows], sem_w))
        pend_w.append(pltpu.async_copy(buf_gm[sel], gm_o.at[rows], sem_w))
        pend_w.append(pltpu.async_copy(buf_mu[sel], mu_o.at[rows], sem_w))
        pend_w.append(pltpu.async_copy(buf_mm[sel], mm_o.at[rows], sem_w))

    for cp in pend_w:
        cp.wait()


@functools.partial(jax.jit, static_argnums=0)
def _sc_gather(h, user_ids, movie_ids, gu_t, gm_t, mu_t, mm_t):
    mesh = plsc.VectorSubcoreMesh(core_axis_name="c", subcore_axis_name="s",
                                  num_cores=NC, num_subcores=NS)
    row = jax.ShapeDtypeStruct((HALF, D), jnp.float32)
    dbuf = [pltpu.VMEM((CHUNK, D), jnp.float32)] * 2
    return pl.kernel(
        functools.partial(_sc_gather_half, h * HALF),
        out_type=[row, row, row, row],
        mesh=mesh,
        scratch_types=[
            pltpu.VMEM((BPW,), jnp.int32),
            pltpu.VMEM((BPW,), jnp.int32),
            dbuf, dbuf, dbuf, dbuf,
            pltpu.SemaphoreType.DMA,
            pltpu.SemaphoreType.DMA,
        ],
    )(user_ids, movie_ids, gu_t, gm_t, mu_t, mm_t)


BT = 4096


def _tc_dense_body(h, prev, gu, gm, mu, mm, w1t, b1, w2t, b2, woutt, bb, out):
    del h, prev
    h = jnp.concatenate([mu[...], mm[...]], axis=1)
    h1 = jnp.maximum(
        jnp.dot(h, w1t[...], preferred_element_type=jnp.float32)
        + b1[...], 0.0)
    h2 = jnp.maximum(
        jnp.dot(h1, w2t[...], preferred_element_type=jnp.float32)
        + b2[...], 0.0)
    cat = jnp.concatenate([gu[...] * gm[...], h2], axis=1)
    o = jnp.dot(cat, woutt[...], preferred_element_type=jnp.float32)
    out[...] = o[:, 0] + bb[0]


@functools.partial(jax.jit, static_argnums=0)
def _tc_dense(h, prev, gu, gm, mu, mm, w1t, b1, w2t, b2, woutt, bb):
    row_spec = pl.BlockSpec((BT, D), lambda i: (i, 0))

    def full(shape):
        return pl.BlockSpec(shape, lambda i: (0, 0))

    nblk = HALF // BT
    off = h * nblk
    grid = (nblk,)
    return pl.pallas_call(
        functools.partial(_tc_dense_body, h),
        grid=grid,
        in_specs=[pl.BlockSpec(memory_space=pl.ANY),
                  row_spec, row_spec, row_spec, row_spec,
                  full((2 * D, 64)), full((1, 64)),
                  full((64, D)), full((1, D)), full((2 * D, 1)),
                  pl.BlockSpec(memory_space=pltpu.SMEM)],
        out_specs=pl.BlockSpec((BT,), lambda i: (i + off,)),
        out_shape=jax.ShapeDtypeStruct((B,), jnp.float32),
        input_output_aliases={0: 0},
    )(prev, gu, gm, mu, mm, w1t, b1, w2t, b2, woutt, bb)


def kernel(user_ids, movie_ids, gmf_user_table, gmf_movie_table,
           mlp_user_table, mlp_movie_table, W1, b1, W2, b2, Wout, bout):
    w1t = W1.T
    w2t = W2.T
    woutt = Wout.T

    out = jnp.zeros((B,), jnp.float32)
    for h in range(2):
        gu, gm, mu, mm = _sc_gather(h, user_ids, movie_ids, gmf_user_table,
                                    gmf_movie_table, mlp_user_table,
                                    mlp_movie_table)
        out = _tc_dense(h, out, gu, gm, mu, mm, w1t, b1.reshape(1, -1),
                        w2t, b2.reshape(1, -1), woutt, bout)
    return out
